# baseline (speedup 1.0000x reference)
import jax
import jax.numpy as jnp
from jax import lax
from jax.experimental import pallas as pl
from jax.experimental.pallas import tpu as pltpu

_B, _H, _W, _D = 32, 32, 32, 1024
_K = 9
_T = 9


def _shift_h(a, di):
    if di == 0:
        return a
    z = jnp.zeros_like(a[:1])
    return jnp.concatenate([a[1:], z], 0) if di == 1 else jnp.concatenate([z, a[:-1]], 0)


def _shift_w(a, dj):
    if dj == 0:
        return a
    z = jnp.zeros_like(a[:, :1])
    return jnp.concatenate([a[:, 1:], z], 1) if dj == 1 else jnp.concatenate([z, a[:, :-1]], 1)


def _affinity_kernel(x_ref, dw_ref, pw_ref, b_ref, out_ref):
    x = x_ref[0].reshape(_H * _W, _D)
    dw = dw_ref[...]
    pw = pw_ref[...]
    w2 = jnp.concatenate([dw[:, t:t + 1] * pw for t in range(_T)], axis=1)
    z = jnp.dot(x, w2, preferred_element_type=jnp.float32,
                precision=lax.Precision.HIGHEST)
    z = z.reshape(_H, _W, _T * _K)
    logits = jnp.zeros((_H, _W, _K), jnp.float32)
    for t in range(_T):
        di, dj = t // 3 - 1, t % 3 - 1
        logits = logits + _shift_w(_shift_h(z[:, :, t * _K:(t + 1) * _K], di), dj)
    logits = logits + b_ref[0][None, None, :]
    m = jnp.max(logits, axis=-1, keepdims=True)
    e = jnp.exp(logits - m)
    out_ref[0] = e / jnp.sum(e, axis=-1, keepdims=True)


def kernel(tok2d, dw_w, pw_w, pw_b):
    dw2 = dw_w.reshape(_D, _T)
    pw2 = pw_w.reshape(_K, _D).T
    b2 = pw_b.reshape(1, _K)
    return pl.pallas_call(
        _affinity_kernel,
        grid=(_B,),
        in_specs=[
            pl.BlockSpec((1, _H, _W, _D), lambda b: (b, 0, 0, 0)),
            pl.BlockSpec((_D, _T), lambda b: (0, 0)),
            pl.BlockSpec((_D, _K), lambda b: (0, 0)),
            pl.BlockSpec((1, _K), lambda b: (0, 0)),
        ],
        out_specs=pl.BlockSpec((1, _H, _W, _K), lambda b: (b, 0, 0, 0)),
        out_shape=jax.ShapeDtypeStruct((_B, _H, _W, _K), jnp.float32),
        compiler_params=pltpu.CompilerParams(
            dimension_semantics=("parallel",),
        ),
    )(tok2d, dw2, pw2, b2)


# speedup vs baseline: 1.6389x; 1.6389x over previous
import jax
import jax.numpy as jnp
from jax import lax
from jax.experimental import pallas as pl
from jax.experimental.pallas import tpu as pltpu

_B, _H, _W, _D = 32, 32, 32, 1024
_K = 9
_T = 9


def _shift_h(a, di):
    if di == 0:
        return a
    z = jnp.zeros_like(a[:1])
    return jnp.concatenate([a[1:], z], 0) if di == 1 else jnp.concatenate([z, a[:-1]], 0)


def _shift_w(a, dj):
    if dj == 0:
        return a
    z = jnp.zeros_like(a[:, :1])
    return jnp.concatenate([a[:, 1:], z], 1) if dj == 1 else jnp.concatenate([z, a[:, :-1]], 1)


def _affinity_kernel(x_ref, dw_ref, pw_ref, b_ref, out_ref):
    x = x_ref[0].reshape(_H * _W, _D)
    dw = dw_ref[...]
    pw = pw_ref[...]
    w2 = jnp.concatenate([dw[:, t:t + 1] * pw for t in range(_T)], axis=1)
    z = jnp.dot(x, w2, preferred_element_type=jnp.float32,
                precision=lax.Precision.DEFAULT)
    z = z.reshape(_H, _W, _T * _K)
    logits = jnp.zeros((_H, _W, _K), jnp.float32)
    for t in range(_T):
        di, dj = t // 3 - 1, t % 3 - 1
        logits = logits + _shift_w(_shift_h(z[:, :, t * _K:(t + 1) * _K], di), dj)
    logits = logits + b_ref[0][None, None, :]
    m = jnp.max(logits, axis=-1, keepdims=True)
    e = jnp.exp(logits - m)
    out_ref[0] = e / jnp.sum(e, axis=-1, keepdims=True)


def kernel(tok2d, dw_w, pw_w, pw_b):
    dw2 = dw_w.reshape(_D, _T)
    pw2 = pw_w.reshape(_K, _D).T
    b2 = pw_b.reshape(1, _K)
    return pl.pallas_call(
        _affinity_kernel,
        grid=(_B,),
        in_specs=[
            pl.BlockSpec((1, _H, _W, _D), lambda b: (b, 0, 0, 0)),
            pl.BlockSpec((_D, _T), lambda b: (0, 0)),
            pl.BlockSpec((_D, _K), lambda b: (0, 0)),
            pl.BlockSpec((1, _K), lambda b: (0, 0)),
        ],
        out_specs=pl.BlockSpec((1, _H, _W, _K), lambda b: (b, 0, 0, 0)),
        out_shape=jax.ShapeDtypeStruct((_B, _H, _W, _K), jnp.float32),
        compiler_params=pltpu.CompilerParams(
            dimension_semantics=("parallel",),
        ),
    )(tok2d, dw2, pw2, b2)


# speedup vs baseline: 2.0862x; 1.2729x over previous
import jax
import jax.numpy as jnp
from jax import lax
from jax.experimental import pallas as pl
from jax.experimental.pallas import tpu as pltpu

_B, _H, _W, _D = 32, 32, 32, 1024
_K = 9
_T = 9
_R = _H * _W


def _affinity_kernel(x_ref, dw_ref, pw_ref, b_ref, out_ref):
    x = x_ref[0].astype(jnp.bfloat16)
    dw = dw_ref[...]
    pw = pw_ref[...]
    w2 = jnp.concatenate([dw[:, t:t + 1] * pw for t in range(_T)],
                         axis=1).astype(jnp.bfloat16)
    z = jnp.dot(x, w2, preferred_element_type=jnp.float32)

    zero_row = jnp.zeros_like(z[:1])
    z_by_dj = {
        0: z,
        1: jnp.concatenate([z[1:], zero_row], 0),
        -1: jnp.concatenate([zero_row, z[:-1]], 0),
    }
    w_of_r = lax.broadcasted_iota(jnp.int32, (_R, _T * _K), 0) & (_W - 1)
    lane = lax.broadcasted_iota(jnp.int32, (_R, _T * _K), 1)

    acc = jnp.zeros((_R, _T * _K), jnp.float32)
    for t in range(_T):
        di, dj = t // 3 - 1, t % 3 - 1
        zt = z_by_dj[dj]
        if di == 1:
            zt = jnp.concatenate([zt[_W:], jnp.zeros_like(zt[:_W])], 0)
        elif di == -1:
            zt = jnp.concatenate([jnp.zeros_like(zt[:_W]), zt[:-_W]], 0)
        ok = (lane >= t * _K) & (lane < (t + 1) * _K)
        if dj == 1:
            ok = ok & (w_of_r < _W - 1)
        elif dj == -1:
            ok = ok & (w_of_r > 0)
        acc = acc + jnp.where(ok, zt, 0.0)

    g = (lax.broadcasted_iota(jnp.int32, (_T * _K, _K), 0) % _K
         == lax.broadcasted_iota(jnp.int32, (_T * _K, _K), 1))
    logits = jnp.dot(acc, g.astype(jnp.float32),
                     preferred_element_type=jnp.float32)
    logits = logits + b_ref[...]
    m = jnp.max(logits, axis=-1, keepdims=True)
    e = jnp.exp(logits - m)
    out_ref[0] = e / jnp.sum(e, axis=-1, keepdims=True)


def kernel(tok2d, dw_w, pw_w, pw_b):
    x = tok2d.reshape(_B, _R, _D)
    dw2 = dw_w.reshape(_D, _T)
    pw2 = pw_w.reshape(_K, _D).T
    b2 = pw_b.reshape(1, _K)
    q = pl.pallas_call(
        _affinity_kernel,
        grid=(_B,),
        in_specs=[
            pl.BlockSpec((1, _R, _D), lambda b: (b, 0, 0)),
            pl.BlockSpec((_D, _T), lambda b: (0, 0)),
            pl.BlockSpec((_D, _K), lambda b: (0, 0)),
            pl.BlockSpec((1, _K), lambda b: (0, 0)),
        ],
        out_specs=pl.BlockSpec((1, _R, _K), lambda b: (b, 0, 0)),
        out_shape=jax.ShapeDtypeStruct((_B, _R, _K), jnp.float32),
        compiler_params=pltpu.CompilerParams(
            dimension_semantics=("parallel",),
        ),
    )(x, dw2, pw2, b2)
    return q.reshape(_B, _H, _W, _K)


# speedup vs baseline: 3.1293x; 1.5000x over previous
import jax
import jax.numpy as jnp
from jax import lax
from jax.experimental import pallas as pl
from jax.experimental.pallas import tpu as pltpu

_B, _H, _W, _D = 32, 32, 32, 1024
_K = 9
_T = 9
_R = _H * _W


def _affinity_kernel(x_ref, w2_ref, b_ref, out_ref):
    x = x_ref[0]
    z = jnp.dot(x, w2_ref[...],
                preferred_element_type=jnp.float32)

    zero_row = jnp.zeros_like(z[:1])
    z_by_dj = {
        0: z,
        1: jnp.concatenate([z[1:], zero_row], 0),
        -1: jnp.concatenate([zero_row, z[:-1]], 0),
    }
    w_of_r = lax.broadcasted_iota(jnp.int32, (_R, _T * _K), 0) & (_W - 1)
    lane = lax.broadcasted_iota(jnp.int32, (_R, _T * _K), 1)

    acc = jnp.zeros((_R, _T * _K), jnp.float32)
    for t in range(_T):
        di, dj = t // 3 - 1, t % 3 - 1
        zt = z_by_dj[dj]
        if di == 1:
            zt = jnp.concatenate([zt[_W:], jnp.zeros_like(zt[:_W])], 0)
        elif di == -1:
            zt = jnp.concatenate([jnp.zeros_like(zt[:_W]), zt[:-_W]], 0)
        ok = (lane >= t * _K) & (lane < (t + 1) * _K)
        if dj == 1:
            ok = ok & (w_of_r < _W - 1)
        elif dj == -1:
            ok = ok & (w_of_r > 0)
        acc = acc + jnp.where(ok, zt, 0.0)

    g = (lax.broadcasted_iota(jnp.int32, (_T * _K, _K), 0) % _K
         == lax.broadcasted_iota(jnp.int32, (_T * _K, _K), 1))
    logits = jnp.dot(acc, g.astype(jnp.float32),
                     preferred_element_type=jnp.float32)
    logits = logits + b_ref[...]
    m = jnp.max(logits, axis=-1, keepdims=True)
    e = jnp.exp(logits - m)
    out_ref[0] = e / jnp.sum(e, axis=-1, keepdims=True)


def kernel(tok2d, dw_w, pw_w, pw_b):
    x = tok2d.reshape(_B, _R, _D)
    dw2 = dw_w.reshape(_D, _T)
    pw2 = pw_w.reshape(_K, _D).T
    w2 = (dw2[:, :, None] * pw2[:, None, :]).reshape(_D, _T * _K)
    b2 = pw_b.reshape(1, _K)
    q = pl.pallas_call(
        _affinity_kernel,
        grid=(_B,),
        in_specs=[
            pl.BlockSpec((1, _R, _D), lambda b: (b, 0, 0)),
            pl.BlockSpec((_D, _T * _K), lambda b: (0, 0)),
            pl.BlockSpec((1, _K), lambda b: (0, 0)),
        ],
        out_specs=pl.BlockSpec((1, _R, _K), lambda b: (b, 0, 0)),
        out_shape=jax.ShapeDtypeStruct((_B, _R, _K), jnp.float32),
        compiler_params=pltpu.CompilerParams(
            dimension_semantics=("parallel",),
        ),
    )(x, w2, b2)
    return q.reshape(_B, _H, _W, _K)


# speedup vs baseline: 3.5593x; 1.1374x over previous
import jax
import jax.numpy as jnp
from jax import lax
from jax.experimental import pallas as pl
from jax.experimental.pallas import tpu as pltpu

_B, _H, _W, _D = 32, 32, 32, 1024
_K = 9
_T = 9
_G = 2
_R = _G * _H * _W


def _affinity_kernel(x_ref, w2_ref, b_ref, out_ref):
    x = x_ref[...].reshape(_R, _D)
    z = jnp.dot(x, w2_ref[...],
                preferred_element_type=jnp.float32)

    zero_row = jnp.zeros_like(z[:1])
    z_by_dj = {
        0: z,
        1: jnp.concatenate([z[1:], zero_row], 0),
        -1: jnp.concatenate([zero_row, z[:-1]], 0),
    }
    row = lax.broadcasted_iota(jnp.int32, (_R, _T * _K), 0)
    w_of_r = row & (_W - 1)
    h_of_r = (row // _W) & (_H - 1)
    lane = lax.broadcasted_iota(jnp.int32, (_R, _T * _K), 1)

    acc = jnp.zeros((_R, _T * _K), jnp.float32)
    for t in range(_T):
        di, dj = t // 3 - 1, t % 3 - 1
        zt = z_by_dj[dj]
        if di == 1:
            zt = jnp.concatenate([zt[_W:], jnp.zeros_like(zt[:_W])], 0)
        elif di == -1:
            zt = jnp.concatenate([jnp.zeros_like(zt[:_W]), zt[:-_W]], 0)
        ok = (lane >= t * _K) & (lane < (t + 1) * _K)
        if dj == 1:
            ok = ok & (w_of_r < _W - 1)
        elif dj == -1:
            ok = ok & (w_of_r > 0)
        if di == 1:
            ok = ok & (h_of_r < _H - 1)
        elif di == -1:
            ok = ok & (h_of_r > 0)
        acc = acc + jnp.where(ok, zt, 0.0)

    g = (lax.broadcasted_iota(jnp.int32, (_T * _K, _K), 0) % _K
         == lax.broadcasted_iota(jnp.int32, (_T * _K, _K), 1))
    logits = jnp.dot(acc, g.astype(jnp.float32),
                     preferred_element_type=jnp.float32)
    logits = logits + b_ref[...]
    m = jnp.max(logits, axis=-1, keepdims=True)
    e = jnp.exp(logits - m)
    out_ref[...] = (e / jnp.sum(e, axis=-1, keepdims=True)).reshape(
        _G, _H * _W, _K)


def kernel(tok2d, dw_w, pw_w, pw_b):
    x = tok2d.reshape(_B, _H * _W, _D)
    dw2 = dw_w.reshape(_D, _T)
    pw2 = pw_w.reshape(_K, _D).T
    w2 = (dw2[:, :, None] * pw2[:, None, :]).reshape(_D, _T * _K)
    b2 = pw_b.reshape(1, _K)
    q = pl.pallas_call(
        _affinity_kernel,
        grid=(_B // _G,),
        in_specs=[
            pl.BlockSpec((_G, _H * _W, _D), lambda b: (b, 0, 0)),
            pl.BlockSpec((_D, _T * _K), lambda b: (0, 0)),
            pl.BlockSpec((1, _K), lambda b: (0, 0)),
        ],
        out_specs=pl.BlockSpec((_G, _H * _W, _K), lambda b: (b, 0, 0)),
        out_shape=jax.ShapeDtypeStruct((_B, _H * _W, _K), jnp.float32),
        compiler_params=pltpu.CompilerParams(
            dimension_semantics=("parallel",),
        ),
    )(x, w2, b2)
    return q.reshape(_B, _H, _W, _K)


# speedup vs baseline: 3.6751x; 1.0325x over previous
import jax
import jax.numpy as jnp
from jax import lax
from jax.experimental import pallas as pl
from jax.experimental.pallas import tpu as pltpu

_B, _H, _W, _D = 32, 32, 32, 1024
_K = 9
_T = 9
_G = 4
_R = _G * _H * _W


def _affinity_kernel(x_ref, w2_ref, b_ref, out_ref):
    x = x_ref[...].reshape(_R, _D)
    z = jnp.dot(x, w2_ref[...],
                preferred_element_type=jnp.float32)

    zero_row = jnp.zeros_like(z[:1])
    z_by_dj = {
        0: z,
        1: jnp.concatenate([z[1:], zero_row], 0),
        -1: jnp.concatenate([zero_row, z[:-1]], 0),
    }
    row = lax.broadcasted_iota(jnp.int32, (_R, _T * _K), 0)
    w_of_r = row & (_W - 1)
    h_of_r = (row // _W) & (_H - 1)
    lane = lax.broadcasted_iota(jnp.int32, (_R, _T * _K), 1)

    acc = jnp.zeros((_R, _T * _K), jnp.float32)
    for t in range(_T):
        di, dj = t // 3 - 1, t % 3 - 1
        zt = z_by_dj[dj]
        if di == 1:
            zt = jnp.concatenate([zt[_W:], jnp.zeros_like(zt[:_W])], 0)
        elif di == -1:
            zt = jnp.concatenate([jnp.zeros_like(zt[:_W]), zt[:-_W]], 0)
        ok = (lane >= t * _K) & (lane < (t + 1) * _K)
        if dj == 1:
            ok = ok & (w_of_r < _W - 1)
        elif dj == -1:
            ok = ok & (w_of_r > 0)
        if di == 1:
            ok = ok & (h_of_r < _H - 1)
        elif di == -1:
            ok = ok & (h_of_r > 0)
        acc = acc + jnp.where(ok, zt, 0.0)

    g = (lax.broadcasted_iota(jnp.int32, (_T * _K, _K), 0) % _K
         == lax.broadcasted_iota(jnp.int32, (_T * _K, _K), 1))
    logits = jnp.dot(acc, g.astype(jnp.float32),
                     preferred_element_type=jnp.float32)
    logits = logits + b_ref[...]
    m = jnp.max(logits, axis=-1, keepdims=True)
    e = jnp.exp(logits - m)
    out_ref[...] = (e / jnp.sum(e, axis=-1, keepdims=True)).reshape(
        _G, _H * _W, _K)


def kernel(tok2d, dw_w, pw_w, pw_b):
    x = tok2d.reshape(_B, _H * _W, _D)
    dw2 = dw_w.reshape(_D, _T)
    pw2 = pw_w.reshape(_K, _D).T
    w2 = (dw2[:, :, None] * pw2[:, None, :]).reshape(_D, _T * _K)
    b2 = pw_b.reshape(1, _K)
    q = pl.pallas_call(
        _affinity_kernel,
        grid=(_B // _G,),
        in_specs=[
            pl.BlockSpec((_G, _H * _W, _D), lambda b: (b, 0, 0)),
            pl.BlockSpec((_D, _T * _K), lambda b: (0, 0)),
            pl.BlockSpec((1, _K), lambda b: (0, 0)),
        ],
        out_specs=pl.BlockSpec((_G, _H * _W, _K), lambda b: (b, 0, 0)),
        out_shape=jax.ShapeDtypeStruct((_B, _H * _W, _K), jnp.float32),
        compiler_params=pltpu.CompilerParams(
            dimension_semantics=("parallel",),
        ),
    )(x, w2, b2)
    return q.reshape(_B, _H, _W, _K)
